# baseline (speedup 1.0000x reference)
import functools

import jax
import jax.numpy as jnp
from jax import lax
from jax.experimental import pallas as pl
from jax.experimental.pallas import tpu as pltpu
from jax.experimental.pallas import tpu_sc as plsc

NC = 2
NS = 16
NW = NC * NS
C = 128
L = 16


def _make_kernel(n, d, max_z, ec_dim, nfull, tail, direct):
    mesh = plsc.VectorSubcoreMesh(
        core_axis_name="c", subcore_axis_name="s", num_cores=NC, num_subcores=NS
    )
    q, rem = divmod(nfull, NW)
    kmax = q + (1 if rem else 0)
    smax = max((kmax + (1 if tail else 0)) * C, C)
    zrows = 8
    zp = (max_z + zrows - 1) // zrows * zrows
    ztiles = zp // zrows

    @functools.partial(
        pl.kernel,
        out_type=jax.ShapeDtypeStruct((n, d), jnp.float32),
        mesh=mesh,
        scratch_types=[
            pltpu.VMEM((smax,), jnp.int32),
            pltpu.VMEM((C, d), jnp.float32),
            pltpu.VMEM((C, d), jnp.float32),
            pltpu.VMEM((zrows, d), jnp.float32),
            pltpu.VMEM((zrows, d), jnp.float32),
            pltpu.VMEM((ec_dim, d), jnp.float32),
            pltpu.MemorySpace.VMEM_SHARED((zp, d), jnp.float32),
            pltpu.SemaphoreType.DMA,
            pltpu.SemaphoreType.DMA,
            pltpu.SemaphoreType.DMA,
            pltpu.SemaphoreType.DMA,
            pltpu.SemaphoreType.DMA,
        ],
    )
    def fused_k(ec_hbm, clt_hbm, emb_hbm, idx_hbm, out_hbm,
                idx_v, rows0, rows1, ecemb_v, tab_v, clt_v, table_sp,
                sg0, sg1, ss0, ss1, si):
        sid = lax.axis_index("s")
        wid = sid * NC + lax.axis_index("c")
        nk = jnp.where(wid < rem, q + 1, q)
        start = wid * q + jnp.minimum(wid, rem)
        base = start * C

        if direct:
            wstart = jnp.minimum(base, n - smax)
            off = base - wstart
        else:
            wstart = base
            off = 0
        idx_desc = pltpu.make_async_copy(
            idx_hbm.at[pl.ds(wstart, smax)], idx_v, si
        )
        idx_desc.start()

        @pl.when(sid < ztiles)
        def _():
            zs = pl.multiple_of(sid * zrows, zrows)
            pltpu.sync_copy(clt_hbm, clt_v)
            pltpu.sync_copy(emb_hbm.at[pl.ds(zs, zrows)], tab_v)
            pltpu.sync_copy(ec_hbm.at[pl.ds(zs, zrows)], ecemb_v)
            nec = (ec_dim + L - 1) // L
            for g in range(d // L):
                clt_slices = [clt_v[e, pl.ds(g * L, L)] for e in range(ec_dim)]
                for r in range(zrows):
                    ec_chunks = [ecemb_v[r, pl.ds(c * L, L)] for c in range(nec)]
                    acc = tab_v[r, pl.ds(g * L, L)]
                    for e in range(ec_dim):
                        acc = acc + ec_chunks[e // L][e % L] * clt_slices[e]
                    tab_v[r, pl.ds(g * L, L)] = acc
            pltpu.sync_copy(tab_v, table_sp.at[pl.ds(zs, zrows)])

        plsc.subcore_barrier()

        rows = (rows0, rows1)
        sg = (sg0, sg1)
        ss = (ss0, ss1)

        idx_desc.wait()

        def gather_desc(j, b):
            return pltpu.make_async_copy(
                table_sp.at[idx_v.at[pl.ds(off + j * C, C)]], rows[b], sg[b]
            )

        def store_desc(j, b):
            return pltpu.make_async_copy(
                rows[b], out_hbm.at[pl.ds(base + j * C, C)], ss[b]
            )

        @pl.when(nk > 0)
        def _():
            gather_desc(0, 0).start()

        def handle(j, b):
            @pl.when(j < nk)
            def _():
                gather_desc(j, b).wait()
                store_desc(j, b).start()

                @pl.when(j + 1 < nk)
                def _():
                    @pl.when(j >= 1)
                    def _():
                        store_desc(j - 1, 1 - b).wait()

                    gather_desc(j + 1, 1 - b).start()

        def pair(g, carry):
            handle(2 * g, 0)
            handle(2 * g + 1, 1)
            return carry

        lax.fori_loop(0, (kmax + 1) // 2, pair, 0)

        for b in (0, 1):
            @pl.when(nk > b)
            def _(b=b):
                jl = nk - 1 - ((nk - 1 - b) % 2)
                store_desc(jl, b).wait()

        if tail > 0:
            @pl.when(wid == NW - 1)
            def _():
                if direct:
                    tail_off = nfull * C - wstart
                    tdesc = pltpu.make_async_copy(
                        table_sp.at[idx_v.at[pl.ds(tail_off, tail)]],
                        rows0.at[pl.ds(0, tail)],
                        sg0,
                    )
                else:
                    tdesc = gather_desc(q, 0)
                tdesc.start()
                tdesc.wait()
                pltpu.sync_copy(
                    rows0.at[pl.ds(0, tail)],
                    out_hbm.at[pl.ds(nfull * C, tail)],
                )

    return fused_k


def kernel(atomic_numbers, electron_config, emb_table, config_linear):
    n = atomic_numbers.shape[0]
    max_z, ec_dim = electron_config.shape
    d = emb_table.shape[1]

    zp = (max_z + 7) // 8 * 8
    ec_wide = jnp.zeros((zp, d), jnp.float32).at[:max_z, :ec_dim].set(electron_config)
    emb_pad = jnp.zeros((zp, d), jnp.float32).at[:max_z].set(emb_table)
    clt = config_linear.T

    nfull, tail = divmod(n, C)
    q, rem = divmod(nfull, NW)
    kmax = q + (1 if rem else 0)
    smax = max((kmax + (1 if tail else 0)) * C, C)
    idx = atomic_numbers.astype(jnp.int32)
    direct = (n % 8 == 0) and (n >= smax)
    if not direct:
        last_start = ((NW - 1) * q + min(NW - 1, rem)) * C
        npad = max(last_start + smax, nfull * C + (C if tail else 0))
        idx = jnp.zeros((npad,), jnp.int32).at[:n].set(idx)

    fused_k = _make_kernel(n, d, max_z, ec_dim, nfull, tail, direct)
    return fused_k(ec_wide, clt, emb_pad, idx)


# speedup vs baseline: 1.0476x; 1.0476x over previous
import functools

import jax
import jax.numpy as jnp
from jax import lax
from jax.experimental import pallas as pl
from jax.experimental.pallas import tpu as pltpu
from jax.experimental.pallas import tpu_sc as plsc

NC = 2
NS = 16
NW = NC * NS
C = 128
G = 2
B = C * G


def _combine_body(ec_ref, clt_ref, emb_ref, out_ref):
    out_ref[...] = (
        jnp.dot(ec_ref[...], clt_ref[...], preferred_element_type=jnp.float32)
        + emb_ref[...]
    )


def _build_combined(ec_pad, clt_pad, emb_pad):
    zp, d = emb_pad.shape
    return pl.pallas_call(
        _combine_body,
        out_shape=jax.ShapeDtypeStruct((zp, d), jnp.float32),
    )(ec_pad, clt_pad, emb_pad)


def _make_gather(n, d, zp, nfull, tail, direct):
    mesh = plsc.VectorSubcoreMesh(
        core_axis_name="c", subcore_axis_name="s", num_cores=NC, num_subcores=NS
    )
    q, rem = divmod(nfull, NW)
    kmax = q + (1 if rem else 0)
    smax = max((kmax + (1 if tail else 0)) * B, B)

    @functools.partial(
        pl.kernel,
        out_type=jax.ShapeDtypeStruct((n, d), jnp.float32),
        mesh=mesh,
        scratch_types=[
            pltpu.VMEM((smax,), jnp.int32),
            pltpu.VMEM((B, d), jnp.float32),
            pltpu.VMEM((B, d), jnp.float32),
            pltpu.MemorySpace.VMEM_SHARED((zp, d), jnp.float32),
            pltpu.SemaphoreType.DMA,
            pltpu.SemaphoreType.DMA,
            pltpu.SemaphoreType.DMA,
            pltpu.SemaphoreType.DMA,
        ],
    )
    def gather_k(table_hbm, idx_hbm, out_hbm, idx_v, rows0, rows1, table_sp,
                 sg0, sg1, ss0, ss1):
        wid = lax.axis_index("s") * NC + lax.axis_index("c")

        @pl.when(lax.axis_index("s") == 0)
        def _():
            pltpu.sync_copy(table_hbm, table_sp)

        plsc.subcore_barrier()
        nk = jnp.where(wid < rem, q + 1, q)
        start = wid * q + jnp.minimum(wid, rem)
        base = start * B

        rows = (rows0, rows1)
        sg = (sg0, sg1)
        ss = (ss0, ss1)

        if direct:
            wstart = jnp.minimum(base, n - smax)
            off = base - wstart
        else:
            wstart = base
            off = 0
        pltpu.sync_copy(idx_hbm.at[pl.ds(wstart, smax)], idx_v)

        def gather_descs(j, b):
            return [
                pltpu.make_async_copy(
                    table_sp.at[idx_v.at[pl.ds(off + j * B + p * C, C)]],
                    rows[b].at[pl.ds(p * C, C)],
                    sg[b],
                )
                for p in range(G)
            ]

        def store_desc(j, b):
            return pltpu.make_async_copy(
                rows[b], out_hbm.at[pl.ds(base + j * B, B)], ss[b]
            )

        @pl.when(nk > 0)
        def _():
            for dsc in gather_descs(0, 0):
                dsc.start()

        def handle(j, b):
            @pl.when(j < nk)
            def _():
                for dsc in gather_descs(j, b):
                    dsc.wait()
                store_desc(j, b).start()

                @pl.when(j + 1 < nk)
                def _():
                    @pl.when(j >= 1)
                    def _():
                        store_desc(j - 1, 1 - b).wait()

                    for dsc in gather_descs(j + 1, 1 - b):
                        dsc.start()

        def pair(g, carry):
            handle(2 * g, 0)
            handle(2 * g + 1, 1)
            return carry

        lax.fori_loop(0, (kmax + 1) // 2, pair, 0)

        for b in (0, 1):
            @pl.when(nk > b)
            def _(b=b):
                jl = nk - 1 - ((nk - 1 - b) % 2)
                store_desc(jl, b).wait()

        if tail > 0:
            pieces = []
            done = 0
            while done < tail:
                pieces.append((done, min(C, tail - done)))
                done += pieces[-1][1]

            @pl.when(wid == NW - 1)
            def _():
                tail_off = (nfull * B - wstart) if direct else q * B

                def tdesc(o, sz):
                    return pltpu.make_async_copy(
                        table_sp.at[idx_v.at[pl.ds(tail_off + o, sz)]],
                        rows0.at[pl.ds(o, sz)],
                        sg0,
                    )

                for o, sz in pieces:
                    tdesc(o, sz).start()
                for o, sz in pieces:
                    tdesc(o, sz).wait()
                pltpu.sync_copy(
                    rows0.at[pl.ds(0, tail)],
                    out_hbm.at[pl.ds(nfull * B, tail)],
                )

    return gather_k


def kernel(atomic_numbers, electron_config, emb_table, config_linear):
    n = atomic_numbers.shape[0]
    max_z, ec_dim = electron_config.shape
    d = emb_table.shape[1]

    zp = (max_z + 7) // 8 * 8
    kp = 128
    ec_pad = jnp.zeros((zp, kp), jnp.float32).at[:max_z, :ec_dim].set(electron_config)
    clt_pad = jnp.zeros((kp, d), jnp.float32).at[:ec_dim, :].set(config_linear.T)
    emb_pad = jnp.zeros((zp, d), jnp.float32).at[:max_z, :].set(emb_table)

    combined = _build_combined(ec_pad, clt_pad, emb_pad)

    nfull, tail = divmod(n, B)
    q, rem = divmod(nfull, NW)
    kmax = q + (1 if rem else 0)
    smax = max((kmax + (1 if tail else 0)) * B, B)
    idx = atomic_numbers.astype(jnp.int32)
    direct = (n % 8 == 0) and (n >= smax)
    if not direct:
        last_start = ((NW - 1) * q + min(NW - 1, rem)) * B
        npad = max(last_start + smax, nfull * B + (B if tail else 0))
        idx = jnp.zeros((npad,), jnp.int32).at[:n].set(idx)

    gather_k = _make_gather(n, d, zp, nfull, tail, direct)
    return gather_k(combined, idx)
